# baseline (speedup 1.0000x reference)
import functools

import jax
import jax.numpy as jnp
import numpy as np
from jax import lax
from jax.experimental import pallas as pl
from jax.experimental.pallas import tpu as pltpu
from jax.experimental.pallas import tpu_sc as plsc

NC = 2
NS = 16
NW = NC * NS
L = 16
CH = 40
G = 5


def _dot(a, b):
    return lax.dot_general(a, b, (((1,), (0,)), ((), ())),
                           precision=lax.Precision.DEFAULT,
                           preferred_element_type=jnp.float32)


def _ln(y, g, b):
    mu = jnp.mean(y, axis=-1, keepdims=True)
    var = jnp.mean((y - mu) ** 2, axis=-1, keepdims=True)
    return (y - mu) / jnp.sqrt(var + 1e-5) * g + b


def _pre_body(x_ref, a_ref, b_ref, p_ref, q_ref):
    xb = x_ref[...]
    p_ref[...] = _dot(xb, a_ref[...])
    q_ref[...] = _dot(xb, b_ref[...])


def _tc_pre(x, A, B, bn=1000):
    n, d = x.shape
    return pl.pallas_call(
        _pre_body,
        grid=(n // bn,),
        in_specs=[pl.BlockSpec((bn, d), lambda i: (i, 0)),
                  pl.BlockSpec((d, d), lambda i: (0, 0)),
                  pl.BlockSpec((d, d), lambda i: (0, 0))],
        out_specs=[pl.BlockSpec((bn, d), lambda i: (i, 0)),
                   pl.BlockSpec((bn, d), lambda i: (i, 0))],
        out_shape=[jax.ShapeDtypeStruct((n, d), jnp.float32)] * 2,
        compiler_params=pltpu.CompilerParams(
            dimension_semantics=("parallel",)),
    )(x, A, B)


def _edge_mlp_body(s_ref, e_ref, c_ref, b0_ref, w1_ref, b1_ref, w2_ref,
                   b2_ref, g_ref, bb_ref, o_ref):
    h = (s_ref[...].astype(jnp.float32) + _dot(e_ref[...], c_ref[...])
         + b0_ref[...])
    h = jnp.maximum(h, 0.0)
    h = jnp.maximum(_dot(h, w1_ref[...]) + b1_ref[...], 0.0)
    y = _dot(h, w2_ref[...]) + b2_ref[...]
    o_ref[...] = _ln(y, g_ref[...], bb_ref[...])


def _tc_edge_mlp(S, e, C, b0, W1, b1, W2, b2, g, bb, be=1000):
    E, d = S.shape
    wspec = pl.BlockSpec((d, d), lambda i: (0, 0))
    vspec = pl.BlockSpec((1, d), lambda i: (0, 0))
    return pl.pallas_call(
        _edge_mlp_body,
        grid=(E // be,),
        in_specs=[pl.BlockSpec((be, d), lambda i: (i, 0)),
                  pl.BlockSpec((be, d), lambda i: (i, 0)),
                  wspec, vspec, wspec, vspec, wspec, vspec, vspec, vspec],
        out_specs=pl.BlockSpec((be, d), lambda i: (i, 0)),
        out_shape=jax.ShapeDtypeStruct((E, d), jnp.float32),
        compiler_params=pltpu.CompilerParams(
            dimension_semantics=("parallel",)),
    )(S, e, C, b0, W1, b1, W2, b2, g, bb)


def _node_body(x_ref, s00_ref, s01_ref, s10_ref, s11_ref, cnt_ref, ua_ref,
               ub_ref, b0_ref, w1_ref, b1_ref, w2_ref, b2_ref, g_ref, bb_ref,
               o_ref):
    s = (s00_ref[...] + s01_ref[...]) + (s10_ref[...] + s11_ref[...])
    aggr = s / jnp.maximum(cnt_ref[...], 1.0)
    h = _dot(x_ref[...], ua_ref[...]) + _dot(aggr, ub_ref[...]) + b0_ref[...]
    h = jnp.maximum(h, 0.0)
    h = jnp.maximum(_dot(h, w1_ref[...]) + b1_ref[...], 0.0)
    y = _dot(h, w2_ref[...]) + b2_ref[...]
    o_ref[...] = _ln(y, g_ref[...], bb_ref[...])


def _tc_node(x, sums, cnt, Ua, Ub, b0, W1, b1, W2, b2, g, bb, bn=1000):
    n, d = x.shape
    wspec = pl.BlockSpec((d, d), lambda i: (0, 0))
    vspec = pl.BlockSpec((1, d), lambda i: (0, 0))
    nspec = pl.BlockSpec((bn, d), lambda i: (i, 0))
    cspec = pl.BlockSpec((bn, 1), lambda i: (i, 0))
    return pl.pallas_call(
        _node_body,
        grid=(n // bn,),
        in_specs=[nspec, nspec, nspec, nspec, nspec, cspec,
                  wspec, wspec, vspec, wspec, vspec, wspec, vspec, vspec,
                  vspec],
        out_specs=nspec,
        out_shape=jax.ShapeDtypeStruct((n, d), jnp.float32),
        compiler_params=pltpu.CompilerParams(
            dimension_semantics=("parallel",)),
    )(x, sums[0], sums[1], sums[2], sums[3], cnt,
      Ua, Ub, b0, W1, b1, W2, b2, g, bb)


def _sc_gather_sum(P, Q, src2, dst2):
    n, d = P.shape
    dw = d // 2
    ew = src2.shape[1]
    nch = ew // CH
    E = NW * ew
    mesh = plsc.VectorSubcoreMesh(core_axis_name="c", subcore_axis_name="s",
                                  num_cores=NC, num_subcores=NS)

    @functools.partial(
        pl.kernel,
        out_type=jax.ShapeDtypeStruct((E, dw), jnp.int32),
        mesh=mesh,
        compiler_params=pltpu.CompilerParams(needs_layout_passes=False),
        scratch_types=(
            [pltpu.VMEM((ew,), jnp.int32)] * 2
            + [pltpu.VMEM((CH, d), jnp.float32)] * (2 * G)
            + [pltpu.VMEM((CH, dw), jnp.int32)] * G
            + [pltpu.SemaphoreType.DMA] * (2 * G)
        ),
    )
    def k(p_hbm, q_hbm, src_hbm, dst_hbm, out_hbm, *sc):
        di_v, si_v = sc[0], sc[1]
        bufd = sc[2:2 + G]
        bufq = sc[2 + G:2 + 2 * G]
        sbuf = sc[2 + 2 * G:2 + 3 * G]
        gsem = sc[2 + 3 * G:2 + 4 * G]
        wsem = sc[2 + 4 * G:2 + 5 * G]
        wid = lax.axis_index("s") * NC + lax.axis_index("c")
        base = wid * ew
        pltpu.sync_copy(dst_hbm.at[wid], di_v)
        pltpu.sync_copy(src_hbm.at[wid], si_v)

        def fire(b, c):
            pltpu.async_copy(p_hbm.at[di_v.at[pl.ds(c * CH, CH)]], bufd[b],
                             gsem[b])
            pltpu.async_copy(q_hbm.at[si_v.at[pl.ds(c * CH, CH)]], bufq[b],
                             gsem[b])

        for b in range(G):
            fire(b, b)

        def body(kk, carry):
            for b in range(G):
                c = kk * G + b
                pltpu.make_async_copy(p_hbm.at[di_v.at[pl.ds(0, CH)]],
                                      bufd[b], gsem[b]).wait()
                pltpu.make_async_copy(q_hbm.at[si_v.at[pl.ds(0, CH)]],
                                      bufq[b], gsem[b]).wait()

                @pl.when(kk > 0)
                def _():
                    pltpu.make_async_copy(
                        sbuf[b], out_hbm.at[pl.ds(base, CH)], wsem[b]).wait()

                def row(r, carry2):
                    for t in range(dw // L):
                        lo = pl.ds((2 * t) * L, L)
                        hi = pl.ds((2 * t + 1) * L, L)
                        slo = bufd[b][r, lo] + bufq[b][r, lo]
                        shi = bufd[b][r, hi] + bufq[b][r, hi]
                        pk = plsc.pack(slo, shi,
                                       format=plsc.PackFormat.INTERLEAVED)
                        sbuf[b][r, pl.ds(t * L, L)] = plsc.bitcast(
                            pk, jnp.int32)
                    return carry2

                lax.fori_loop(0, CH, row, 0)
                pltpu.async_copy(sbuf[b],
                                 out_hbm.at[pl.ds(base + c * CH, CH)],
                                 wsem[b])

                @pl.when(c + G < nch)
                def _():
                    fire(b, c + G)
            return carry

        lax.fori_loop(0, nch // G, body, 0)
        for b in range(G):
            pltpu.make_async_copy(sbuf[b], out_hbm.at[pl.ds(base, CH)],
                                  wsem[b]).wait()

    return k(P, Q, src2, dst2)


HR = 80


def _sc_scatter_add(m, dst3, dst2, n):
    E, d = m.shape
    nch = dst3.shape[1]
    ew = E // NW
    ewp = dst2.shape[1]
    zc = CH
    nzc = n // zc
    kmax = -(-nzc // NS)
    nhv = ew // L
    rem = ew % L
    mesh = plsc.VectorSubcoreMesh(core_axis_name="c", subcore_axis_name="s",
                                  num_cores=NC, num_subcores=NS)

    @functools.partial(
        pl.kernel,
        out_type=(jax.ShapeDtypeStruct((NC, n, d), jnp.float32),
                  jax.ShapeDtypeStruct((NC, HR, d), jnp.float32)),
        mesh=mesh,
        compiler_params=pltpu.CompilerParams(needs_layout_passes=False),
        scratch_types=[
            pltpu.VMEM((CH,), jnp.int32),
            pltpu.VMEM((CH,), jnp.int32),
            pltpu.VMEM((CH, d), jnp.float32),
            pltpu.VMEM((CH, d), jnp.float32),
            pltpu.VMEM((ewp,), jnp.int32),
            pltpu.VMEM((HR, d), jnp.float32),
            pltpu.VMEM((HR,), jnp.int32),
            pltpu.SemaphoreType.DMA,
            pltpu.SemaphoreType.DMA,
            pltpu.VMEM_SHARED((n, d), jnp.float32),
            pltpu.VMEM_SHARED((HR, d), jnp.float32),
        ],
    )
    def k(m_hbm, dst_hbm, dst2_hbm, out_hbm, cnt_hbm, ib0, ib1, buf0, buf1,
          dst1, hist, iot, sem0, sem1, acc, acc_cnt):
        cid = lax.axis_index("c")
        sid = lax.axis_index("s")
        wid = sid * NC + cid
        base = wid * (nch * CH)
        ibs = (ib0, ib1)
        bufs = (buf0, buf1)
        sems = (sem0, sem1)

        def frow(r, carry):
            for j in range(d // L):
                buf0[r, pl.ds(j * L, L)] = jnp.zeros((L,), jnp.float32)
            return carry

        lax.fori_loop(0, CH, frow, 0)

        def hrow(r, carry):
            for j in range(d // L):
                hist[r, pl.ds(j * L, L)] = jnp.zeros((L,), jnp.float32)
            return carry

        lax.fori_loop(0, HR, hrow, 0)
        for v in range(HR // L):
            iot[pl.ds(v * L, L)] = lax.iota(jnp.int32, L) + v * L

        for kk in range(kmax):
            c = sid + kk * NS

            @pl.when(c < nzc)
            def _():
                pltpu.sync_copy(buf0, acc.at[pl.ds(c * zc, zc)])

        @pl.when(sid < HR // CH)
        def _():
            pltpu.sync_copy(buf0, acc_cnt.at[pl.ds(sid * CH, CH)])

        plsc.subcore_barrier()
        pltpu.sync_copy(dst2_hbm.at[wid], dst1)

        def fire_idx(b, c):
            pltpu.async_copy(dst_hbm.at[wid, c], ibs[b], sems[b])

        def drain_idx(b):
            pltpu.make_async_copy(dst_hbm.at[wid, 0], ibs[b], sems[b]).wait()

        for b in range(2):
            fire_idx(b, b)
            pltpu.async_copy(m_hbm.at[pl.ds(base + b * CH, CH)], bufs[b],
                             sems[b])

        def pair(kk, carry):
            for b in range(2):
                c = 2 * kk + b
                drain_idx(b)
                pltpu.make_async_copy(
                    m_hbm.at[pl.ds(base, CH)], bufs[b], sems[b]).wait()
                pltpu.sync_copy(bufs[b], acc.at[ibs[b]], add=True)

                @pl.when(c + 2 < nch)
                def _():
                    fire_idx(b, c + 2)
                    pltpu.async_copy(
                        m_hbm.at[pl.ds(base + (c + 2) * CH, CH)],
                        bufs[b], sems[b])
            return carry

        lax.fori_loop(0, nch // 2, pair, 0)
        if nch % 2:
            b = (nch - 1) % 2
            drain_idx(b)
            pltpu.make_async_copy(
                m_hbm.at[pl.ds(base, CH)], bufs[b], sems[b]).wait()
            pltpu.sync_copy(bufs[b], acc.at[ibs[b]], add=True)

        ones = jnp.ones((L,), jnp.float32)

        def hvec(v, carry):
            idx = dst1[pl.ds(v * L, L)]
            row = lax.shift_right_logical(idx, 7)
            col = lax.bitwise_and(idx, 127)
            plsc.addupdate_scatter(hist, [row, col], ones)
            return carry

        lax.fori_loop(0, nhv, hvec, 0)
        if rem:
            idx = dst1[pl.ds(nhv * L, L)]
            row = lax.shift_right_logical(idx, 7)
            col = lax.bitwise_and(idx, 127)
            msk = lax.iota(jnp.int32, L) < rem
            plsc.addupdate_scatter(hist, [row, col], ones, mask=msk)
        pltpu.sync_copy(hist, acc_cnt.at[iot], add=True)

        plsc.subcore_barrier()
        for kk in range(kmax):
            c = sid + kk * NS

            @pl.when(c < nzc)
            def _():
                pltpu.sync_copy(acc.at[pl.ds(c * zc, zc)],
                                out_hbm.at[cid, pl.ds(c * zc, zc)])

        @pl.when(sid < HR // CH)
        def _():
            pltpu.sync_copy(acc_cnt.at[pl.ds(sid * CH, CH)],
                            cnt_hbm.at[cid, pl.ds(sid * CH, CH)])

    return k(m, dst3, dst2)


def kernel(x, edge_index, edge_attr, params):
    n, d = x.shape
    E = edge_index.shape[1]
    assert d == 128 and E % (NW * CH) == 0 and n % (NS * 5) == 0

    src = edge_index[0].astype(jnp.int32)
    dst = edge_index[1].astype(jnp.int32)

    pm, pn = params["msg"], params["node"]
    W0, W1, W2 = pm["Ws"]
    b0, b1, b2 = [b.reshape(1, d) for b in pm["bs"]]
    g, bb = pm["g"].reshape(1, d), pm["b"].reshape(1, d)
    A, B, C = W0[:d], W0[d:2 * d], W0[2 * d:]

    P, Q = _tc_pre(x, A, B)

    perm = np.empty((d,), np.int32)
    for t in range(d // 32):
        for i in range(16):
            perm[32 * t + 2 * i] = 32 * t + i
            perm[32 * t + 2 * i + 1] = 32 * t + 16 + i
    Cp, b0p, W1p = C[:, perm], b0[:, perm], W1[perm, :]

    nsl = 2
    eh = E // nsl
    ew = eh // NW
    ewp = -(-ew // L) * L
    sums, cnts = [], []
    for k in range(nsl):
        sl = slice(k * eh, (k + 1) * eh)
        src2 = src[sl].reshape(NW, ew)
        dst2 = dst[sl].reshape(NW, ew)
        dst2p = jnp.pad(dst2, ((0, 0), (0, ewp - ew)))
        dst3 = dst[sl].reshape(NW, ew // CH, CH)
        Si = _sc_gather_sum(P, Q, src2, dst2)
        S = lax.bitcast_convert_type(Si, jnp.bfloat16).reshape(eh, d)
        m = _tc_edge_mlp(S, edge_attr[sl], Cp, b0p, W1p, b1, W2, b2, g, bb)
        psum, pcnt = _sc_scatter_add(m, dst3, dst2p, n)
        sums.extend([psum[0], psum[1]])
        cnts.append(pcnt[0] + pcnt[1])

    cnt = (cnts[0] + cnts[1]).reshape(HR * d)[:n].reshape(n, 1)

    U0, V1, V2 = pn["Ws"]
    c0, c1, c2 = [b.reshape(1, d) for b in pn["bs"]]
    gn, bn = pn["g"].reshape(1, d), pn["b"].reshape(1, d)
    Ua, Ub = U0[:d], U0[d:]
    x_out = _tc_node(x, sums, cnt, Ua, Ub, c0, V1, c1, V2, c2, gn, bn)
    return (x_out, edge_attr)


# speedup vs baseline: 1.8024x; 1.8024x over previous
import functools

import jax
import jax.numpy as jnp
import numpy as np
from jax import lax
from jax.experimental import pallas as pl
from jax.experimental.pallas import tpu as pltpu
from jax.experimental.pallas import tpu_sc as plsc

NC = 2
NS = 16
NW = NC * NS
L = 16
CH = 40
G = 5


def _dot(a, b):
    return lax.dot_general(a, b, (((1,), (0,)), ((), ())),
                           precision=lax.Precision.DEFAULT,
                           preferred_element_type=jnp.float32)


def _ln(y, g, b):
    mu = jnp.mean(y, axis=-1, keepdims=True)
    var = jnp.mean((y - mu) ** 2, axis=-1, keepdims=True)
    return (y - mu) / jnp.sqrt(var + 1e-5) * g + b


def _pre_body(x_ref, a_ref, b_ref, p_ref, q_ref):
    xb = x_ref[...]
    p_ref[...] = _dot(xb, a_ref[...])
    q_ref[...] = _dot(xb, b_ref[...])


def _tc_pre(x, A, B, bn=1000):
    n, d = x.shape
    return pl.pallas_call(
        _pre_body,
        grid=(n // bn,),
        in_specs=[pl.BlockSpec((bn, d), lambda i: (i, 0)),
                  pl.BlockSpec((d, d), lambda i: (0, 0)),
                  pl.BlockSpec((d, d), lambda i: (0, 0))],
        out_specs=[pl.BlockSpec((bn, d), lambda i: (i, 0)),
                   pl.BlockSpec((bn, d), lambda i: (i, 0))],
        out_shape=[jax.ShapeDtypeStruct((n, d), jnp.float32)] * 2,
        compiler_params=pltpu.CompilerParams(
            dimension_semantics=("parallel",)),
    )(x, A, B)


def _edge_mlp_body(s_ref, e_ref, c_ref, b0_ref, w1_ref, b1_ref, w2_ref,
                   b2_ref, g_ref, bb_ref, o_ref):
    w = s_ref[...]
    sa = lax.bitcast_convert_type(lax.shift_left(w, 16), jnp.float32)
    sb = lax.bitcast_convert_type(
        lax.bitwise_and(w, jnp.int32(-65536)), jnp.float32)
    s2 = jnp.concatenate([sa, sb], axis=1)
    h = s2 + _dot(e_ref[...], c_ref[...]) + b0_ref[...]
    h = jnp.maximum(h, 0.0)
    h = jnp.maximum(_dot(h, w1_ref[...]) + b1_ref[...], 0.0)
    y = _dot(h, w2_ref[...]) + b2_ref[...]
    o_ref[...] = _ln(y, g_ref[...], bb_ref[...])


def _tc_edge_mlp(S, e, C, b0, W1, b1, W2, b2, g, bb, be=1000):
    E, dw = S.shape
    d = 2 * dw
    wspec = pl.BlockSpec((d, d), lambda i: (0, 0))
    vspec = pl.BlockSpec((1, d), lambda i: (0, 0))
    return pl.pallas_call(
        _edge_mlp_body,
        grid=(E // be,),
        in_specs=[pl.BlockSpec((be, dw), lambda i: (i, 0)),
                  pl.BlockSpec((be, d), lambda i: (i, 0)),
                  wspec, vspec, wspec, vspec, wspec, vspec, vspec, vspec],
        out_specs=pl.BlockSpec((be, d), lambda i: (i, 0)),
        out_shape=jax.ShapeDtypeStruct((E, d), jnp.float32),
        compiler_params=pltpu.CompilerParams(
            dimension_semantics=("parallel",)),
    )(S, e, C, b0, W1, b1, W2, b2, g, bb)


def _node_body(x_ref, s00_ref, s01_ref, s10_ref, s11_ref, cnt_ref, ua_ref,
               ub_ref, b0_ref, w1_ref, b1_ref, w2_ref, b2_ref, g_ref, bb_ref,
               o_ref):
    s = (s00_ref[...] + s01_ref[...]) + (s10_ref[...] + s11_ref[...])
    aggr = s / jnp.maximum(cnt_ref[...], 1.0)
    h = _dot(x_ref[...], ua_ref[...]) + _dot(aggr, ub_ref[...]) + b0_ref[...]
    h = jnp.maximum(h, 0.0)
    h = jnp.maximum(_dot(h, w1_ref[...]) + b1_ref[...], 0.0)
    y = _dot(h, w2_ref[...]) + b2_ref[...]
    o_ref[...] = _ln(y, g_ref[...], bb_ref[...])


def _tc_node(x, sums, cnt, Ua, Ub, b0, W1, b1, W2, b2, g, bb, bn=1000):
    n, d = x.shape
    wspec = pl.BlockSpec((d, d), lambda i: (0, 0))
    vspec = pl.BlockSpec((1, d), lambda i: (0, 0))
    nspec = pl.BlockSpec((bn, d), lambda i: (i, 0))
    cspec = pl.BlockSpec((bn, 1), lambda i: (i, 0))
    return pl.pallas_call(
        _node_body,
        grid=(n // bn,),
        in_specs=[nspec, nspec, nspec, nspec, nspec, cspec,
                  wspec, wspec, vspec, wspec, vspec, wspec, vspec, vspec,
                  vspec],
        out_specs=nspec,
        out_shape=jax.ShapeDtypeStruct((n, d), jnp.float32),
        compiler_params=pltpu.CompilerParams(
            dimension_semantics=("parallel",)),
    )(x, sums[0], sums[1], sums[2], sums[3], cnt,
      Ua, Ub, b0, W1, b1, W2, b2, g, bb)


def _sc_gather_sum(P, Q, src2, dst2):
    n, d = P.shape
    dw = d // 2
    ew = src2.shape[1]
    nch = ew // CH
    E = NW * ew
    mesh = plsc.VectorSubcoreMesh(core_axis_name="c", subcore_axis_name="s",
                                  num_cores=NC, num_subcores=NS)

    @functools.partial(
        pl.kernel,
        out_type=jax.ShapeDtypeStruct((E, dw), jnp.int32),
        mesh=mesh,
        compiler_params=pltpu.CompilerParams(needs_layout_passes=False),
        scratch_types=(
            [pltpu.VMEM((ew,), jnp.int32)] * 2
            + [pltpu.VMEM((CH, d), jnp.float32)] * (2 * G)
            + [pltpu.VMEM((CH, dw), jnp.int32)] * G
            + [pltpu.SemaphoreType.DMA] * (2 * G)
        ),
    )
    def k(p_hbm, q_hbm, src_hbm, dst_hbm, out_hbm, *sc):
        di_v, si_v = sc[0], sc[1]
        bufd = sc[2:2 + G]
        bufq = sc[2 + G:2 + 2 * G]
        sbuf = sc[2 + 2 * G:2 + 3 * G]
        gsem = sc[2 + 3 * G:2 + 4 * G]
        wsem = sc[2 + 4 * G:2 + 5 * G]
        wid = lax.axis_index("s") * NC + lax.axis_index("c")
        base = wid * ew
        pltpu.sync_copy(dst_hbm.at[wid], di_v)
        pltpu.sync_copy(src_hbm.at[wid], si_v)

        def fire(b, c):
            pltpu.async_copy(p_hbm.at[di_v.at[pl.ds(c * CH, CH)]], bufd[b],
                             gsem[b])
            pltpu.async_copy(q_hbm.at[si_v.at[pl.ds(c * CH, CH)]], bufq[b],
                             gsem[b])

        for b in range(G):
            fire(b, b)

        def body(kk, carry):
            for b in range(G):
                c = kk * G + b
                pltpu.make_async_copy(p_hbm.at[di_v.at[pl.ds(0, CH)]],
                                      bufd[b], gsem[b]).wait()
                pltpu.make_async_copy(q_hbm.at[si_v.at[pl.ds(0, CH)]],
                                      bufq[b], gsem[b]).wait()

                @pl.when(kk > 0)
                def _():
                    pltpu.make_async_copy(
                        sbuf[b], out_hbm.at[pl.ds(base, CH)], wsem[b]).wait()

                def row(r, carry2):
                    for t in range(dw // L):
                        lo = pl.ds((2 * t) * L, L)
                        hi = pl.ds((2 * t + 1) * L, L)
                        slo = bufd[b][r, lo] + bufq[b][r, lo]
                        shi = bufd[b][r, hi] + bufq[b][r, hi]
                        pk = plsc.pack(slo, shi,
                                       format=plsc.PackFormat.INTERLEAVED)
                        sbuf[b][r, pl.ds(t * L, L)] = plsc.bitcast(
                            pk, jnp.int32)
                    return carry2

                lax.fori_loop(0, CH, row, 0)
                pltpu.async_copy(sbuf[b],
                                 out_hbm.at[pl.ds(base + c * CH, CH)],
                                 wsem[b])

                @pl.when(c + G < nch)
                def _():
                    fire(b, c + G)
            return carry

        lax.fori_loop(0, nch // G, body, 0)
        for b in range(G):
            pltpu.make_async_copy(sbuf[b], out_hbm.at[pl.ds(base, CH)],
                                  wsem[b]).wait()

    return k(P, Q, src2, dst2)


HR = 80


def _sc_scatter_add(m, dst3, dst2, n):
    E, d = m.shape
    nch = dst3.shape[1]
    ew = dst2.shape[1]
    zc = CH
    nzc = n // zc
    kmax = -(-nzc // NS)
    nhv = ew // L
    rem = ew % L
    assert ew >= L
    mesh = plsc.VectorSubcoreMesh(core_axis_name="c", subcore_axis_name="s",
                                  num_cores=NC, num_subcores=NS)

    @functools.partial(
        pl.kernel,
        out_type=(jax.ShapeDtypeStruct((NC, n, d), jnp.float32),
                  jax.ShapeDtypeStruct((NC, HR, d), jnp.float32)),
        mesh=mesh,
        compiler_params=pltpu.CompilerParams(needs_layout_passes=False),
        scratch_types=[
            pltpu.VMEM((CH,), jnp.int32),
            pltpu.VMEM((CH,), jnp.int32),
            pltpu.VMEM((CH, d), jnp.float32),
            pltpu.VMEM((CH, d), jnp.float32),
            pltpu.VMEM((ew,), jnp.int32),
            pltpu.VMEM((HR, d), jnp.float32),
            pltpu.VMEM((HR,), jnp.int32),
            pltpu.SemaphoreType.DMA,
            pltpu.SemaphoreType.DMA,
            pltpu.VMEM_SHARED((n, d), jnp.float32),
            pltpu.VMEM_SHARED((HR, d), jnp.float32),
        ],
    )
    def k(m_hbm, dst_hbm, dst2_hbm, out_hbm, cnt_hbm, ib0, ib1, buf0, buf1,
          dst1, hist, iot, sem0, sem1, acc, acc_cnt):
        cid = lax.axis_index("c")
        sid = lax.axis_index("s")
        wid = sid * NC + cid
        base = wid * (nch * CH)
        ibs = (ib0, ib1)
        bufs = (buf0, buf1)
        sems = (sem0, sem1)

        def frow(r, carry):
            for j in range(d // L):
                buf0[r, pl.ds(j * L, L)] = jnp.zeros((L,), jnp.float32)
            return carry

        lax.fori_loop(0, CH, frow, 0)

        def hrow(r, carry):
            for j in range(d // L):
                hist[r, pl.ds(j * L, L)] = jnp.zeros((L,), jnp.float32)
            return carry

        lax.fori_loop(0, HR, hrow, 0)
        for v in range(HR // L):
            iot[pl.ds(v * L, L)] = lax.iota(jnp.int32, L) + v * L

        for kk in range(kmax):
            c = sid + kk * NS

            @pl.when(c < nzc)
            def _():
                pltpu.sync_copy(buf0, acc.at[pl.ds(c * zc, zc)])

        @pl.when(sid < HR // CH)
        def _():
            pltpu.sync_copy(buf0, acc_cnt.at[pl.ds(sid * CH, CH)])

        plsc.subcore_barrier()
        pltpu.sync_copy(dst2_hbm.at[wid], dst1)

        def fire_idx(b, c):
            pltpu.async_copy(dst_hbm.at[wid, c], ibs[b], sems[b])

        def drain_idx(b):
            pltpu.make_async_copy(dst_hbm.at[wid, 0], ibs[b], sems[b]).wait()

        for b in range(2):
            fire_idx(b, b)
            pltpu.async_copy(m_hbm.at[pl.ds(base + b * CH, CH)], bufs[b],
                             sems[b])

        def pair(kk, carry):
            for b in range(2):
                c = 2 * kk + b
                drain_idx(b)
                pltpu.make_async_copy(
                    m_hbm.at[pl.ds(base, CH)], bufs[b], sems[b]).wait()
                pltpu.sync_copy(bufs[b], acc.at[ibs[b]], add=True)

                @pl.when(c + 2 < nch)
                def _():
                    fire_idx(b, c + 2)
                    pltpu.async_copy(
                        m_hbm.at[pl.ds(base + (c + 2) * CH, CH)],
                        bufs[b], sems[b])
            return carry

        lax.fori_loop(0, nch // 2, pair, 0)
        if nch % 2:
            b = (nch - 1) % 2
            drain_idx(b)
            pltpu.make_async_copy(
                m_hbm.at[pl.ds(base, CH)], bufs[b], sems[b]).wait()
            pltpu.sync_copy(bufs[b], acc.at[ibs[b]], add=True)

        ones = jnp.ones((L,), jnp.float32)

        def hvec(v, carry):
            idx = dst1[pl.ds(v * L, L)]
            row = lax.shift_right_logical(idx, 7)
            col = lax.bitwise_and(idx, 127)
            plsc.addupdate_scatter(hist, [row, col], ones)
            return carry

        lax.fori_loop(0, nhv, hvec, 0)
        if rem:
            idx = dst1[pl.ds(ew - L, L)]
            row = lax.shift_right_logical(idx, 7)
            col = lax.bitwise_and(idx, 127)
            msk = lax.iota(jnp.int32, L) >= (L - rem)
            plsc.addupdate_scatter(hist, [row, col], ones, mask=msk)
        pltpu.sync_copy(hist, acc_cnt.at[iot], add=True)

        plsc.subcore_barrier()
        for kk in range(kmax):
            c = sid + kk * NS

            @pl.when(c < nzc)
            def _():
                pltpu.sync_copy(acc.at[pl.ds(c * zc, zc)],
                                out_hbm.at[cid, pl.ds(c * zc, zc)])

        @pl.when(sid < HR // CH)
        def _():
            pltpu.sync_copy(acc_cnt.at[pl.ds(sid * CH, CH)],
                            cnt_hbm.at[cid, pl.ds(sid * CH, CH)])

    return k(m, dst3, dst2)


def kernel(x, edge_index, edge_attr, params):
    n, d = x.shape
    E = edge_index.shape[1]
    assert d == 128 and E % (NW * CH) == 0 and n % (NS * 5) == 0

    src = edge_index[0].astype(jnp.int32)
    dst = edge_index[1].astype(jnp.int32)

    pm, pn = params["msg"], params["node"]
    W0, W1, W2 = pm["Ws"]
    b0, b1, b2 = [b.reshape(1, d) for b in pm["bs"]]
    g, bb = pm["g"].reshape(1, d), pm["b"].reshape(1, d)
    A, B, C = W0[:d], W0[d:2 * d], W0[2 * d:]

    P, Q = _tc_pre(x, A, B)

    perm = np.empty((d,), np.int32)
    for t in range(d // 32):
        for i in range(16):
            perm[16 * t + i] = 32 * t + i
            perm[d // 2 + 16 * t + i] = 32 * t + 16 + i
    Cp, b0p, W1p = C[:, perm], b0[:, perm], W1[perm, :]

    nsl = 2
    eh = E // nsl
    ew = eh // NW
    sums, cnts = [], []
    for k in range(nsl):
        sl = slice(k * eh, (k + 1) * eh)
        src2 = src[sl].reshape(NW, ew)
        dst2 = dst[sl].reshape(NW, ew)
        dst3 = dst[sl].reshape(NW, ew // CH, CH)
        Si = _sc_gather_sum(P, Q, src2, dst2)
        m = _tc_edge_mlp(Si, edge_attr[sl], Cp, b0p, W1p, b1, W2, b2, g, bb)
        psum, pcnt = _sc_scatter_add(m, dst3, dst2, n)
        sums.extend([psum[0], psum[1]])
        cnts.append(pcnt[0] + pcnt[1])

    cnt = (cnts[0] + cnts[1]).reshape(HR * d)[:n].reshape(n, 1)

    U0, V1, V2 = pn["Ws"]
    c0, c1, c2 = [b.reshape(1, d) for b in pn["bs"]]
    gn, bn = pn["g"].reshape(1, d), pn["b"].reshape(1, d)
    Ua, Ub = U0[:d], U0[d:]
    x_out = _tc_node(x, sums, cnt, Ua, Ub, c0, V1, c1, V2, c2, gn, bn)
    return (x_out, edge_attr)


# speedup vs baseline: 1.8758x; 1.0407x over previous
import functools

import jax
import jax.numpy as jnp
import numpy as np
from jax import lax
from jax.experimental import pallas as pl
from jax.experimental.pallas import tpu as pltpu
from jax.experimental.pallas import tpu_sc as plsc

NC = 2
NS = 16
NW = NC * NS
L = 16
CH = 40
G = 5


def _dot(a, b):
    return lax.dot_general(a, b, (((1,), (0,)), ((), ())),
                           precision=lax.Precision.DEFAULT,
                           preferred_element_type=jnp.float32)


def _ln(y, g, b):
    mu = jnp.mean(y, axis=-1, keepdims=True)
    var = jnp.mean((y - mu) ** 2, axis=-1, keepdims=True)
    return (y - mu) / jnp.sqrt(var + 1e-5) * g + b


def _pre_body(x_ref, a_ref, b_ref, p_ref, q_ref):
    xb = x_ref[...]
    p_ref[...] = _dot(xb, a_ref[...])
    q_ref[...] = _dot(xb, b_ref[...])


def _tc_pre(x, A, B, bn=1000):
    n, d = x.shape
    return pl.pallas_call(
        _pre_body,
        grid=(n // bn,),
        in_specs=[pl.BlockSpec((bn, d), lambda i: (i, 0)),
                  pl.BlockSpec((d, d), lambda i: (0, 0)),
                  pl.BlockSpec((d, d), lambda i: (0, 0))],
        out_specs=[pl.BlockSpec((bn, d), lambda i: (i, 0)),
                   pl.BlockSpec((bn, d), lambda i: (i, 0))],
        out_shape=[jax.ShapeDtypeStruct((n, d), jnp.float32)] * 2,
        compiler_params=pltpu.CompilerParams(
            dimension_semantics=("parallel",)),
    )(x, A, B)


def _edge_mlp_body(s_ref, e_ref, c_ref, b0_ref, w1_ref, b1_ref, w2_ref,
                   b2_ref, g_ref, bb_ref, o_ref):
    w = s_ref[...]
    sa = lax.bitcast_convert_type(lax.shift_left(w, 16), jnp.float32)
    sb = lax.bitcast_convert_type(
        lax.bitwise_and(w, jnp.int32(-65536)), jnp.float32)
    s2 = jnp.concatenate([sa, sb], axis=1)
    h = s2 + _dot(e_ref[...], c_ref[...]) + b0_ref[...]
    h = jnp.maximum(h, 0.0)
    h = jnp.maximum(_dot(h, w1_ref[...]) + b1_ref[...], 0.0)
    y = _dot(h, w2_ref[...]) + b2_ref[...]
    o_ref[...] = _ln(y, g_ref[...], bb_ref[...])


def _tc_edge_mlp(S, e, C, b0, W1, b1, W2, b2, g, bb, be=1000):
    E, dw = S.shape
    d = 2 * dw
    wspec = pl.BlockSpec((d, d), lambda i: (0, 0))
    vspec = pl.BlockSpec((1, d), lambda i: (0, 0))
    return pl.pallas_call(
        _edge_mlp_body,
        grid=(E // be,),
        in_specs=[pl.BlockSpec((be, dw), lambda i: (i, 0)),
                  pl.BlockSpec((be, d), lambda i: (i, 0)),
                  wspec, vspec, wspec, vspec, wspec, vspec, vspec, vspec],
        out_specs=pl.BlockSpec((be, d), lambda i: (i, 0)),
        out_shape=jax.ShapeDtypeStruct((E, d), jnp.float32),
        compiler_params=pltpu.CompilerParams(
            dimension_semantics=("parallel",)),
    )(S, e, C, b0, W1, b1, W2, b2, g, bb)


def _node_body(nsum, *refs):
    (x_ref, srefs, cnt_ref) = refs[0], refs[1:1 + nsum], refs[1 + nsum]
    (ua_ref, ub_ref, b0_ref, w1_ref, b1_ref, w2_ref, b2_ref, g_ref,
     bb_ref, o_ref) = refs[2 + nsum:]
    s = srefs[0][...]
    for r in srefs[1:]:
        s = s + r[...]
    aggr = s / jnp.maximum(cnt_ref[...], 1.0)
    h = _dot(x_ref[...], ua_ref[...]) + _dot(aggr, ub_ref[...]) + b0_ref[...]
    h = jnp.maximum(h, 0.0)
    h = jnp.maximum(_dot(h, w1_ref[...]) + b1_ref[...], 0.0)
    y = _dot(h, w2_ref[...]) + b2_ref[...]
    o_ref[...] = _ln(y, g_ref[...], bb_ref[...])


def _tc_node(x, sums, cnt, Ua, Ub, b0, W1, b1, W2, b2, g, bb, bn=1000):
    n, d = x.shape
    wspec = pl.BlockSpec((d, d), lambda i: (0, 0))
    vspec = pl.BlockSpec((1, d), lambda i: (0, 0))
    nspec = pl.BlockSpec((bn, d), lambda i: (i, 0))
    cspec = pl.BlockSpec((bn, 1), lambda i: (i, 0))
    return pl.pallas_call(
        functools.partial(_node_body, len(sums)),
        grid=(n // bn,),
        in_specs=([nspec] + [nspec] * len(sums) + [cspec]
                  + [wspec, wspec, vspec, wspec, vspec, wspec, vspec, vspec,
                     vspec]),
        out_specs=nspec,
        out_shape=jax.ShapeDtypeStruct((n, d), jnp.float32),
        compiler_params=pltpu.CompilerParams(
            dimension_semantics=("parallel",)),
    )(x, *sums, cnt, Ua, Ub, b0, W1, b1, W2, b2, g, bb)


def _sc_gather_sum(P, Q, src2, dst2):
    n, d = P.shape
    dw = d // 2
    ew = src2.shape[1]
    nch = ew // CH
    E = NW * ew
    mesh = plsc.VectorSubcoreMesh(core_axis_name="c", subcore_axis_name="s",
                                  num_cores=NC, num_subcores=NS)

    @functools.partial(
        pl.kernel,
        out_type=jax.ShapeDtypeStruct((E, dw), jnp.int32),
        mesh=mesh,
        compiler_params=pltpu.CompilerParams(needs_layout_passes=False),
        scratch_types=(
            [pltpu.VMEM((ew,), jnp.int32)] * 2
            + [pltpu.VMEM((CH, d), jnp.float32)] * (2 * G)
            + [pltpu.VMEM((CH, dw), jnp.int32)] * G
            + [pltpu.SemaphoreType.DMA] * (2 * G)
        ),
    )
    def k(p_hbm, q_hbm, src_hbm, dst_hbm, out_hbm, *sc):
        di_v, si_v = sc[0], sc[1]
        bufd = sc[2:2 + G]
        bufq = sc[2 + G:2 + 2 * G]
        sbuf = sc[2 + 2 * G:2 + 3 * G]
        gsem = sc[2 + 3 * G:2 + 4 * G]
        wsem = sc[2 + 4 * G:2 + 5 * G]
        wid = lax.axis_index("s") * NC + lax.axis_index("c")
        base = wid * ew
        pltpu.sync_copy(dst_hbm.at[wid], di_v)
        pltpu.sync_copy(src_hbm.at[wid], si_v)

        def fire(b, c):
            pltpu.async_copy(p_hbm.at[di_v.at[pl.ds(c * CH, CH)]], bufd[b],
                             gsem[b])
            pltpu.async_copy(q_hbm.at[si_v.at[pl.ds(c * CH, CH)]], bufq[b],
                             gsem[b])

        for b in range(G):
            fire(b, b)

        def body(kk, carry):
            for b in range(G):
                c = kk * G + b
                pltpu.make_async_copy(p_hbm.at[di_v.at[pl.ds(0, CH)]],
                                      bufd[b], gsem[b]).wait()
                pltpu.make_async_copy(q_hbm.at[si_v.at[pl.ds(0, CH)]],
                                      bufq[b], gsem[b]).wait()

                @pl.when(kk > 0)
                def _():
                    pltpu.make_async_copy(
                        sbuf[b], out_hbm.at[pl.ds(base, CH)], wsem[b]).wait()

                def row(r, carry2):
                    for t in range(dw // L):
                        lo = pl.ds((2 * t) * L, L)
                        hi = pl.ds((2 * t + 1) * L, L)
                        slo = bufd[b][r, lo] + bufq[b][r, lo]
                        shi = bufd[b][r, hi] + bufq[b][r, hi]
                        pk = plsc.pack(slo, shi,
                                       format=plsc.PackFormat.INTERLEAVED)
                        sbuf[b][r, pl.ds(t * L, L)] = plsc.bitcast(
                            pk, jnp.int32)
                    return carry2

                lax.fori_loop(0, CH, row, 0)
                pltpu.async_copy(sbuf[b],
                                 out_hbm.at[pl.ds(base + c * CH, CH)],
                                 wsem[b])

                @pl.when(c + G < nch)
                def _():
                    fire(b, c + G)
            return carry

        lax.fori_loop(0, nch // G, body, 0)
        for b in range(G):
            pltpu.make_async_copy(sbuf[b], out_hbm.at[pl.ds(base, CH)],
                                  wsem[b]).wait()

    return k(P, Q, src2, dst2)


HR = 80


def _sc_scatter_add(m, dst3, dst2, n):
    E, d = m.shape
    nch = dst3.shape[1]
    ew = dst2.shape[1]
    zc = CH
    nzc = n // zc
    kmax = -(-nzc // NS)
    nhv = ew // L
    rem = ew % L
    assert ew >= L
    mesh = plsc.VectorSubcoreMesh(core_axis_name="c", subcore_axis_name="s",
                                  num_cores=NC, num_subcores=NS)

    @functools.partial(
        pl.kernel,
        out_type=(jax.ShapeDtypeStruct((NC, n, d), jnp.float32),
                  jax.ShapeDtypeStruct((NC, HR, d), jnp.float32)),
        mesh=mesh,
        compiler_params=pltpu.CompilerParams(needs_layout_passes=False),
        scratch_types=[
            pltpu.VMEM((CH,), jnp.int32),
            pltpu.VMEM((CH,), jnp.int32),
            pltpu.VMEM((CH, d), jnp.float32),
            pltpu.VMEM((CH, d), jnp.float32),
            pltpu.VMEM((ew,), jnp.int32),
            pltpu.VMEM((HR, d), jnp.float32),
            pltpu.VMEM((HR,), jnp.int32),
            pltpu.SemaphoreType.DMA,
            pltpu.SemaphoreType.DMA,
            pltpu.VMEM_SHARED((n, d), jnp.float32),
            pltpu.VMEM_SHARED((HR, d), jnp.float32),
        ],
    )
    def k(m_hbm, dst_hbm, dst2_hbm, out_hbm, cnt_hbm, ib0, ib1, buf0, buf1,
          dst1, hist, iot, sem0, sem1, acc, acc_cnt):
        cid = lax.axis_index("c")
        sid = lax.axis_index("s")
        wid = sid * NC + cid
        base = wid * (nch * CH)
        ibs = (ib0, ib1)
        bufs = (buf0, buf1)
        sems = (sem0, sem1)

        def frow(r, carry):
            for j in range(d // L):
                buf0[r, pl.ds(j * L, L)] = jnp.zeros((L,), jnp.float32)
            return carry

        lax.fori_loop(0, CH, frow, 0)

        def hrow(r, carry):
            for j in range(d // L):
                hist[r, pl.ds(j * L, L)] = jnp.zeros((L,), jnp.float32)
            return carry

        lax.fori_loop(0, HR, hrow, 0)
        for v in range(HR // L):
            iot[pl.ds(v * L, L)] = lax.iota(jnp.int32, L) + v * L

        for kk in range(kmax):
            c = sid + kk * NS

            @pl.when(c < nzc)
            def _():
                pltpu.sync_copy(buf0, acc.at[pl.ds(c * zc, zc)])

        @pl.when(sid < HR // CH)
        def _():
            pltpu.sync_copy(buf0, acc_cnt.at[pl.ds(sid * CH, CH)])

        plsc.subcore_barrier()
        pltpu.sync_copy(dst2_hbm.at[wid], dst1)

        def fire_idx(b, c):
            pltpu.async_copy(dst_hbm.at[wid, c], ibs[b], sems[b])

        def drain_idx(b):
            pltpu.make_async_copy(dst_hbm.at[wid, 0], ibs[b], sems[b]).wait()

        for b in range(2):
            fire_idx(b, b)
            pltpu.async_copy(m_hbm.at[pl.ds(base + b * CH, CH)], bufs[b],
                             sems[b])

        def pair(kk, carry):
            for b in range(2):
                c = 2 * kk + b
                drain_idx(b)
                pltpu.make_async_copy(
                    m_hbm.at[pl.ds(base, CH)], bufs[b], sems[b]).wait()
                pltpu.sync_copy(bufs[b], acc.at[ibs[b]], add=True)

                @pl.when(c + 2 < nch)
                def _():
                    fire_idx(b, c + 2)
                    pltpu.async_copy(
                        m_hbm.at[pl.ds(base + (c + 2) * CH, CH)],
                        bufs[b], sems[b])
            return carry

        lax.fori_loop(0, nch // 2, pair, 0)
        if nch % 2:
            b = (nch - 1) % 2
            drain_idx(b)
            pltpu.make_async_copy(
                m_hbm.at[pl.ds(base, CH)], bufs[b], sems[b]).wait()
            pltpu.sync_copy(bufs[b], acc.at[ibs[b]], add=True)

        ones = jnp.ones((L,), jnp.float32)

        def hvec(v, carry):
            idx = dst1[pl.ds(v * L, L)]
            row = lax.shift_right_logical(idx, 7)
            col = lax.bitwise_and(idx, 127)
            plsc.addupdate_scatter(hist, [row, col], ones)
            return carry

        lax.fori_loop(0, nhv, hvec, 0)
        if rem:
            idx = dst1[pl.ds(ew - L, L)]
            row = lax.shift_right_logical(idx, 7)
            col = lax.bitwise_and(idx, 127)
            msk = lax.iota(jnp.int32, L) >= (L - rem)
            plsc.addupdate_scatter(hist, [row, col], ones, mask=msk)
        pltpu.sync_copy(hist, acc_cnt.at[iot], add=True)

        plsc.subcore_barrier()
        for kk in range(kmax):
            c = sid + kk * NS

            @pl.when(c < nzc)
            def _():
                pltpu.sync_copy(acc.at[pl.ds(c * zc, zc)],
                                out_hbm.at[cid, pl.ds(c * zc, zc)])

        @pl.when(sid < HR // CH)
        def _():
            pltpu.sync_copy(acc_cnt.at[pl.ds(sid * CH, CH)],
                            cnt_hbm.at[cid, pl.ds(sid * CH, CH)])

    return k(m, dst3, dst2)


def kernel(x, edge_index, edge_attr, params):
    n, d = x.shape
    E = edge_index.shape[1]
    assert d == 128 and E % (NW * CH) == 0 and n % (NS * 5) == 0

    src = edge_index[0].astype(jnp.int32)
    dst = edge_index[1].astype(jnp.int32)

    pm, pn = params["msg"], params["node"]
    W0, W1, W2 = pm["Ws"]
    b0, b1, b2 = [b.reshape(1, d) for b in pm["bs"]]
    g, bb = pm["g"].reshape(1, d), pm["b"].reshape(1, d)
    A, B, C = W0[:d], W0[d:2 * d], W0[2 * d:]

    P, Q = _tc_pre(x, A, B)

    perm = np.empty((d,), np.int32)
    for t in range(d // 32):
        for i in range(16):
            perm[16 * t + i] = 32 * t + i
            perm[d // 2 + 16 * t + i] = 32 * t + 16 + i
    Cp, b0p, W1p = C[:, perm], b0[:, perm], W1[perm, :]

    nsl = 5
    eh = E // nsl
    ew = eh // NW
    sums, cnts = [], []
    for k in range(nsl):
        sl = slice(k * eh, (k + 1) * eh)
        src2 = src[sl].reshape(NW, ew)
        dst2 = dst[sl].reshape(NW, ew)
        dst3 = dst[sl].reshape(NW, ew // CH, CH)
        Si = _sc_gather_sum(P, Q, src2, dst2)
        m = _tc_edge_mlp(Si, edge_attr[sl], Cp, b0p, W1p, b1, W2, b2, g, bb)
        psum, pcnt = _sc_scatter_add(m, dst3, dst2, n)
        sums.extend([psum[0], psum[1]])
        cnts.append(pcnt[0] + pcnt[1])

    cnt_hr = cnts[0]
    for c_ in cnts[1:]:
        cnt_hr = cnt_hr + c_
    cnt = cnt_hr.reshape(HR * d)[:n].reshape(n, 1)

    U0, V1, V2 = pn["Ws"]
    c0, c1, c2 = [b.reshape(1, d) for b in pn["bs"]]
    gn, bn = pn["g"].reshape(1, d), pn["b"].reshape(1, d)
    Ua, Ub = U0[:d], U0[d:]
    x_out = _tc_node(x, sums, cnt, Ua, Ub, c0, V1, c1, V2, c2, gn, bn)
    return (x_out, edge_attr)
